# baseline (speedup 1.0000x reference)
import functools

import jax
import jax.numpy as jnp
from jax import lax
from jax.experimental import pallas as pl
from jax.experimental.pallas import tpu as pltpu
from jax.experimental.pallas import tpu_sc as plsc

_B, _H, _W, _P = 8, 512, 512, 50000
_HW = _H * _W
_PPAD = 50176
_NP = _PPAD // 4
_CHUNK = 128
_NCHUNK = _NP // _CHUNK
_TP = _B * _PPAD

_sc_mesh = plsc.VectorSubcoreMesh(core_axis_name="c", subcore_axis_name="s")


@functools.partial(
    pl.kernel,
    mesh=_sc_mesh,
    out_type=[
        jax.ShapeDtypeStruct((_TP,), jnp.float32),
        jax.ShapeDtypeStruct((_TP,), jnp.float32),
    ],
    scratch_types=[
        pltpu.VMEM((_NP,), jnp.int32),
        pltpu.VMEM((_NP,), jnp.int32),
        pltpu.VMEM((_NP,), jnp.int32),
        pltpu.VMEM((_NP,), jnp.float32),
        pltpu.SemaphoreType.DMA,
    ],
)
def _sc_gather(table, xa, ya, xb, yb, za, zb, xv, yv, idxv, zv, sem):
    c = lax.axis_index("c")
    s = lax.axis_index("s")
    w = s * 2 + c
    base = pl.multiple_of(w * _NP, _NP)
    bhw = (w // 4) * _HW

    for xh, yh, zout in ((xa, ya, za), (xb, yb, zb)):
        pltpu.sync_copy(xh.at[pl.ds(base, _NP)], xv)
        pltpu.sync_copy(yh.at[pl.ds(base, _NP)], yv)

        def idx_body(i, _):
            off = pl.multiple_of(i * 16, 16)
            y16 = yv[pl.ds(off, 16)]
            x16 = xv[pl.ds(off, 16)]
            idxv[pl.ds(off, 16)] = y16 * _W + x16 + bhw
            return 0

        lax.fori_loop(0, _NP // 16, idx_body, 0)

        def gather_body(j, _):
            off = pl.multiple_of(j * _CHUNK, _CHUNK)
            pltpu.async_copy(
                table.at[idxv.at[pl.ds(off, _CHUNK)]],
                zv.at[pl.ds(off, _CHUNK)],
                sem,
            ).wait()
            return 0

        lax.fori_loop(0, _NCHUNK, gather_body, 0)
        pltpu.sync_copy(zv, zout.at[pl.ds(base, _NP)])


def _tc_loss_body(za_ref, zb_ref, gt_ref, out_ref):
    d = za_ref[...] - zb_ref[...]
    g = gt_ref[...]
    m = jnp.abs(g)
    loss = m * jnp.log(1.0 + jnp.exp(-g * d)) + (1.0 - m) * (d * d)
    out_ref[0, 0] = jnp.sum(loss) * (1.0 / (_B * _P))


_tc_loss = pl.pallas_call(
    _tc_loss_body,
    out_shape=jax.ShapeDtypeStruct((1, 1), jnp.float32),
    out_specs=pl.BlockSpec(memory_space=pltpu.SMEM),
)


def kernel(input, x_A, y_A, x_B, y_B, ground_truth):
    table = input.reshape(-1)
    pad = ((0, 0), (0, _PPAD - _P))
    xa = jnp.pad(x_A, pad).astype(jnp.int32).reshape(-1)
    ya = jnp.pad(y_A, pad).astype(jnp.int32).reshape(-1)
    xb = jnp.pad(x_B, pad).astype(jnp.int32).reshape(-1)
    yb = jnp.pad(y_B, pad).astype(jnp.int32).reshape(-1)
    gt = jnp.pad(ground_truth, pad).reshape(_TP // 128, 128)
    za, zb = _sc_gather(table, xa, ya, xb, yb)
    out = _tc_loss(za.reshape(_TP // 128, 128), zb.reshape(_TP // 128, 128), gt)
    return out[0, 0]


# speedup vs baseline: 2.6429x; 2.6429x over previous
import functools

import jax
import jax.numpy as jnp
from jax import lax
from jax.experimental import pallas as pl
from jax.experimental.pallas import tpu as pltpu
from jax.experimental.pallas import tpu_sc as plsc

_B, _H, _W, _P = 8, 512, 512, 50000
_HW = _H * _W
_PPAD = 50176
_NP = _PPAD // 4
_CHUNK = 128
_NCHUNK = _NP // _CHUNK
_TP = _B * _PPAD

_sc_mesh = plsc.VectorSubcoreMesh(core_axis_name="c", subcore_axis_name="s")


@functools.partial(
    pl.kernel,
    mesh=_sc_mesh,
    out_type=[
        jax.ShapeDtypeStruct((_TP,), jnp.float32),
        jax.ShapeDtypeStruct((_TP,), jnp.float32),
    ],
    scratch_types=[
        pltpu.VMEM((_NP,), jnp.int32),
        pltpu.VMEM((_NP,), jnp.int32),
        pltpu.VMEM((_NP,), jnp.int32),
        pltpu.VMEM((_NP,), jnp.int32),
        pltpu.VMEM((_NP,), jnp.float32),
        pltpu.VMEM((_NP,), jnp.float32),
        pltpu.SemaphoreType.DMA,
    ],
)
def _sc_gather(table, xa, ya, xb, yb, za, zb, xv, yv, idxa, idxb, zav, zbv, sem):
    c = lax.axis_index("c")
    s = lax.axis_index("s")
    w = s * 2 + c
    base = pl.multiple_of(w * _NP, _NP)
    bhw = (w // 4) * _HW

    def make_idx_body(xv_, yv_, idx_):
        def idx_body(i, _):
            off = pl.multiple_of(i * 16, 16)
            y16 = yv_[pl.ds(off, 16)]
            x16 = xv_[pl.ds(off, 16)]
            idx_[pl.ds(off, 16)] = y16 * _W + x16 + bhw
            return 0
        return idx_body

    pltpu.sync_copy(xa.at[pl.ds(base, _NP)], xv)
    pltpu.sync_copy(ya.at[pl.ds(base, _NP)], yv)
    lax.fori_loop(0, _NP // 16, make_idx_body(xv, yv, idxa), 0)
    ca = pltpu.async_copy(table.at[idxa], zav, sem)

    pltpu.sync_copy(xb.at[pl.ds(base, _NP)], xv)
    pltpu.sync_copy(yb.at[pl.ds(base, _NP)], yv)
    lax.fori_loop(0, _NP // 16, make_idx_body(xv, yv, idxb), 0)
    cb = pltpu.async_copy(table.at[idxb], zbv, sem)

    ca.wait()
    cb.wait()
    pltpu.sync_copy(zav, za.at[pl.ds(base, _NP)])
    pltpu.sync_copy(zbv, zb.at[pl.ds(base, _NP)])


def _tc_loss_body(za_ref, zb_ref, gt_ref, out_ref):
    d = za_ref[...] - zb_ref[...]
    g = gt_ref[...]
    m = jnp.abs(g)
    loss = m * jnp.log(1.0 + jnp.exp(-g * d)) + (1.0 - m) * (d * d)
    out_ref[0, 0] = jnp.sum(loss) * (1.0 / (_B * _P))


_tc_loss = pl.pallas_call(
    _tc_loss_body,
    out_shape=jax.ShapeDtypeStruct((1, 1), jnp.float32),
    out_specs=pl.BlockSpec(memory_space=pltpu.SMEM),
)


def kernel(input, x_A, y_A, x_B, y_B, ground_truth):
    table = input.reshape(-1)
    pad = ((0, 0), (0, _PPAD - _P))
    xa = jnp.pad(x_A, pad).astype(jnp.int32).reshape(-1)
    ya = jnp.pad(y_A, pad).astype(jnp.int32).reshape(-1)
    xb = jnp.pad(x_B, pad).astype(jnp.int32).reshape(-1)
    yb = jnp.pad(y_B, pad).astype(jnp.int32).reshape(-1)
    gt = jnp.pad(ground_truth, pad).reshape(_TP // 128, 128)
    za, zb = _sc_gather(table, xa, ya, xb, yb)
    out = _tc_loss(za.reshape(_TP // 128, 128), zb.reshape(_TP // 128, 128), gt)
    return out[0, 0]


# speedup vs baseline: 2.7319x; 1.0337x over previous
import functools

import jax
import jax.numpy as jnp
from jax import lax
from jax.experimental import pallas as pl
from jax.experimental.pallas import tpu as pltpu
from jax.experimental.pallas import tpu_sc as plsc

_B, _H, _W, _P = 8, 512, 512, 50000
_HW = _H * _W
_N = _B * _P
_NW = 32
_NPW = 12512
_NPL = _N - 31 * _NPW

_sc_mesh = plsc.VectorSubcoreMesh(core_axis_name="c", subcore_axis_name="s")


@functools.partial(
    pl.kernel,
    mesh=_sc_mesh,
    out_type=[
        jax.ShapeDtypeStruct((_N,), jnp.float32),
        jax.ShapeDtypeStruct((_N,), jnp.float32),
    ],
    scratch_types=[
        pltpu.VMEM((_NPW,), jnp.int32),
        pltpu.VMEM((_NPW,), jnp.int32),
        pltpu.VMEM((_NPW,), jnp.int32),
        pltpu.VMEM((_NPW,), jnp.int32),
        pltpu.VMEM((_NPW,), jnp.float32),
        pltpu.VMEM((_NPW,), jnp.float32),
        pltpu.SemaphoreType.DMA,
    ],
)
def _sc_gather(table, xa, ya, xb, yb, za, zb, xv, yv, idxa, idxb, zav, zbv, sem):
    c = lax.axis_index("c")
    s = lax.axis_index("s")
    w = s * 2 + c
    base = pl.multiple_of(w * _NPW, _NPW)
    last = w == _NW - 1
    b_lo = base // _P
    bnd = (b_lo + 1) * _P
    lane = lax.broadcasted_iota(jnp.int32, (16,), 0)

    def stage(src, dstv):
        @pl.when(jnp.logical_not(last))
        def _():
            pltpu.sync_copy(src.at[pl.ds(base, _NPW)], dstv)

        @pl.when(last)
        def _():
            pltpu.sync_copy(src.at[pl.ds(base, _NPL)], dstv.at[pl.ds(0, _NPL)])

    def make_idx_body(idx_):
        def idx_body(i, _):
            off = pl.multiple_of(i * 16, 16)
            gid = lane + (base + off)
            b = b_lo + jnp.where(gid >= bnd, 1, 0)
            raw = b * _HW + yv[pl.ds(off, 16)] * _W + xv[pl.ds(off, 16)]
            idx_[pl.ds(off, 16)] = jnp.clip(raw, 0, _B * _HW - 1)
            return 0
        return idx_body

    stage(xa, xv)
    stage(ya, yv)
    lax.fori_loop(0, _NPW // 16, make_idx_body(idxa), 0)
    ca = pltpu.async_copy(table.at[idxa], zav, sem)

    stage(xb, xv)
    stage(yb, yv)
    lax.fori_loop(0, _NPW // 16, make_idx_body(idxb), 0)
    cb = pltpu.async_copy(table.at[idxb], zbv, sem)

    ca.wait()
    cb.wait()

    @pl.when(jnp.logical_not(last))
    def _():
        pltpu.sync_copy(zav, za.at[pl.ds(base, _NPW)])
        pltpu.sync_copy(zbv, zb.at[pl.ds(base, _NPW)])

    @pl.when(last)
    def _():
        pltpu.sync_copy(zav.at[pl.ds(0, _NPL)], za.at[pl.ds(base, _NPL)])
        pltpu.sync_copy(zbv.at[pl.ds(0, _NPL)], zb.at[pl.ds(base, _NPL)])


def _tc_loss_body(za_ref, zb_ref, gt_ref, out_ref):
    d = za_ref[...] - zb_ref[...]
    g = gt_ref[...]
    m = jnp.abs(g)
    loss = m * jnp.log(1.0 + jnp.exp(-g * d)) + (1.0 - m) * (d * d)
    out_ref[0, 0] = jnp.sum(loss) * (1.0 / _N)


_tc_loss = pl.pallas_call(
    _tc_loss_body,
    out_shape=jax.ShapeDtypeStruct((1, 1), jnp.float32),
    out_specs=pl.BlockSpec(memory_space=pltpu.SMEM),
)


def kernel(input, x_A, y_A, x_B, y_B, ground_truth):
    table = input.reshape(-1)
    xa = x_A.astype(jnp.int32).reshape(-1)
    ya = y_A.astype(jnp.int32).reshape(-1)
    xb = x_B.astype(jnp.int32).reshape(-1)
    yb = y_B.astype(jnp.int32).reshape(-1)
    za, zb = _sc_gather(table, xa, ya, xb, yb)
    shape2d = (_N // 128, 128)
    out = _tc_loss(
        za.reshape(shape2d), zb.reshape(shape2d), ground_truth.reshape(shape2d)
    )
    return out[0, 0]
